# baseline (speedup 1.0000x reference)
import functools

import jax
import jax.numpy as jnp
from jax import lax
from jax.experimental import pallas as pl
from jax.experimental.pallas import tpu as pltpu
from jax.experimental.pallas import tpu_sc as plsc

N = 20000
SCORE_THRESH = 0.5
IOU_THRESH = 0.8
MAX_DET = 100

NSUB = 16
LANES = 16
PER_SUB = 1280
CHUNKS = PER_SUB // LANES
NPAD = NSUB * PER_SUB
BUF = PER_SUB + 4 * LANES

NEG = -1e30
BIG = 1e30


def _nms_kernel(x1_h, y1_h, x2_h, y2_h, sc_h, out_h,
                x1_v, y1_v, x2_v, y2_v, sv, cand_v, allc_v, out_v,
                shared):
    s = lax.axis_index("s")
    base = s * PER_SUB
    base_f = base.astype(jnp.float32)
    io = lax.iota(jnp.int32, LANES)
    iof = io.astype(jnp.float32)
    zeros_i = jnp.zeros((LANES,), jnp.int32)
    neg16 = jnp.full((LANES,), NEG, jnp.float32)
    big16 = jnp.full((LANES,), BIG, jnp.float32)
    false16 = io < 0

    pltpu.sync_copy(x1_h.at[pl.ds(base, PER_SUB)], x1_v.at[pl.ds(0, PER_SUB)])
    pltpu.sync_copy(y1_h.at[pl.ds(base, PER_SUB)], y1_v.at[pl.ds(0, PER_SUB)])
    pltpu.sync_copy(x2_h.at[pl.ds(base, PER_SUB)], x2_v.at[pl.ds(0, PER_SUB)])
    pltpu.sync_copy(y2_h.at[pl.ds(base, PER_SUB)], y2_v.at[pl.ds(0, PER_SUB)])
    pltpu.sync_copy(sc_h.at[pl.ds(base, PER_SUB)], sv.at[pl.ds(0, PER_SUB)])

    def compact_chunk(k, cnt):
        sl = pl.ds(k * LANES, LANES)
        v = sv[sl]
        m = v > SCORE_THRESH
        x1k = x1_v[sl]
        y1k = y1_v[sl]
        x2k = x2_v[sl]
        y2k = y2_v[sl]
        dst = pl.ds(cnt, LANES)
        plsc.store_compressed(sv.at[dst], v, mask=m)
        plsc.store_compressed(x1_v.at[dst], x1k, mask=m)
        plsc.store_compressed(y1_v.at[dst], y1k, mask=m)
        plsc.store_compressed(x2_v.at[dst], x2k, mask=m)
        plsc.store_compressed(y2_v.at[dst], y2k, mask=m)
        return cnt + jnp.max(plsc.all_reduce_population_count(m))

    cnt = lax.fori_loop(0, CHUNKS, compact_chunk, jnp.int32(0))
    for j in range(4):
        sv[pl.ds(cnt + j * LANES, LANES)] = neg16
    nchunks = (cnt + LANES - 1) // LANES

    def argmax_chunk(k, carry):
        rmax, ridx = carry
        v = sv[pl.ds(k * LANES, LANES)]
        pos = base_f + (k * LANES).astype(jnp.float32) + iof
        ridx = jnp.where(v > rmax, pos, ridx)
        rmax = jnp.maximum(rmax, v)
        return rmax, ridx

    carry0 = lax.fori_loop(0, nchunks, argmax_chunk, (neg16, big16))

    def round_body(carry):
        t0, rmax, ridx = carry
        smax = jnp.max(rmax)
        lpos = jnp.min(jnp.where(rmax == smax, ridx, big16))
        li = jnp.clip(lpos - base_f, 0.0, float(BUF - 1)).astype(jnp.int32)
        liv = zeros_i + li
        gx1 = plsc.load_gather(x1_v, [liv])
        gy1 = plsc.load_gather(y1_v, [liv])
        gx2 = plsc.load_gather(x2_v, [liv])
        gy2 = plsc.load_gather(y2_v, [liv])
        row = (jnp.where(io == 0, smax, 0.0)
               + jnp.where(io == 1, lpos, 0.0)
               + jnp.where(io == 2, gx1, 0.0)
               + jnp.where(io == 3, gy1, 0.0)
               + jnp.where(io == 4, gx2, 0.0)
               + jnp.where(io == 5, gy2, 0.0))
        cand_v[...] = row
        pltpu.sync_copy(cand_v, shared.at[pl.ds(s * LANES, LANES)])
        plsc.subcore_barrier()
        pltpu.sync_copy(shared, allc_v)
        plsc.subcore_barrier()
        stride = io * LANES
        scv = plsc.load_gather(allc_v, [stride])
        pkv = plsc.load_gather(allc_v, [stride + 1])
        cx1 = plsc.load_gather(allc_v, [stride + 2])
        cy1 = plsc.load_gather(allc_v, [stride + 3])
        cx2 = plsc.load_gather(allc_v, [stride + 4])
        cy2 = plsc.load_gather(allc_v, [stride + 5])
        avail = scv > NEG

        def select(taken, dead, t):
            unproc = avail & jnp.logical_not(taken)
            cmax = jnp.max(jnp.where(unproc, scv, neg16))
            go = cmax > NEG
            hit = unproc & (scv == cmax)
            ckey = jnp.min(jnp.where(hit, pkv, big16))
            selm = hit & (pkv == ckey)
            isdead = jnp.max(jnp.where(selm & dead, 1.0, 0.0)) > 0.5
            cont = go & jnp.logical_not(isdead) & (t < MAX_DET)
            return selm, cmax, go, cont

        selm0, cmax0, go0, cont0 = select(false16, false16, t0)

        def accept_body(c):
            taken, dead, selm, cmax, acc_lanes, a, t, cont = c
            cl = jnp.min(jnp.where(selm, iof, big16)).astype(jnp.int32)
            clv = zeros_i + cl * LANES
            wx1 = plsc.load_gather(allc_v, [clv + 2])
            wy1 = plsc.load_gather(allc_v, [clv + 3])
            wx2 = plsc.load_gather(allc_v, [clv + 4])
            wy2 = plsc.load_gather(allc_v, [clv + 5])
            wrow = (jnp.where(io == 0, wx1, 0.0)
                    + jnp.where(io == 1, wy1, 0.0)
                    + jnp.where(io == 2, wx2, 0.0)
                    + jnp.where(io == 3, wy2, 0.0)
                    + jnp.where(io == 4, cmax, 0.0))
            acc_lanes = jnp.where(io == a, cl, acc_lanes)

            @pl.when(s == 0)
            def _():
                out_v[pl.ds(t * LANES, LANES)] = wrow

            xx1 = jnp.maximum(wx1, cx1)
            yy1 = jnp.maximum(wy1, cy1)
            xx2 = jnp.minimum(wx2, cx2)
            yy2 = jnp.minimum(wy2, cy2)
            inter = jnp.maximum(xx2 - xx1, 0.0) * jnp.maximum(yy2 - yy1, 0.0)
            area_a = (wx2 - wx1) * (wy2 - wy1)
            area_b = (cx2 - cx1) * (cy2 - cy1)
            union = area_a + area_b - inter
            iou = inter / jnp.maximum(union, 1e-9)
            dead = dead | (iou > IOU_THRESH)
            taken = taken | selm
            a = a + 1
            t = t + 1
            selm2, cmax2, _, cont2 = select(taken, dead, t)
            return taken, dead, selm2, cmax2, acc_lanes, a, t, cont2 & False

        taken, dead, selm, cmax, acc_lanes, a, t1, cont = lax.while_loop(
            lambda c: c[-1], accept_body,
            (false16, false16, selm0, cmax0, zeros_i, jnp.int32(0), t0,
             cont0))

        @pl.when(jnp.logical_not(go0) & (s == 0))
        def _():
            def zfill(tt, _):
                out_v[pl.ds(tt * LANES, LANES)] = jnp.zeros((LANES,),
                                                            jnp.float32)
                return 0
            lax.fori_loop(t0, MAX_DET, zfill, 0)

        t_next = jnp.where(go0, t1, MAX_DET)

        def winner_sweep(ai, _):
            cl_a = jnp.max(jnp.where(io == ai, acc_lanes, 0))
            av = zeros_i + cl_a * LANES
            wx1 = plsc.load_gather(allc_v, [av + 2])
            wy1 = plsc.load_gather(allc_v, [av + 3])
            wx2 = plsc.load_gather(allc_v, [av + 4])
            wy2 = plsc.load_gather(allc_v, [av + 5])
            area_a = (wx2 - wx1) * (wy2 - wy1)

            def supp_chunk(k, _):
                sl = pl.ds(k * LANES, LANES)
                v = sv[sl]
                x1k = x1_v[sl]
                y1k = y1_v[sl]
                x2k = x2_v[sl]
                y2k = y2_v[sl]
                xx1 = jnp.maximum(wx1, x1k)
                yy1 = jnp.maximum(wy1, y1k)
                xx2 = jnp.minimum(wx2, x2k)
                yy2 = jnp.minimum(wy2, y2k)
                inter = (jnp.maximum(xx2 - xx1, 0.0)
                         * jnp.maximum(yy2 - yy1, 0.0))
                area_b = (x2k - x1k) * (y2k - y1k)
                union = area_a + area_b - inter
                iou = inter / jnp.maximum(union, 1e-9)
                sv[sl] = jnp.where(iou > IOU_THRESH, neg16, v)
                return 0

            lax.fori_loop(0, nchunks, supp_chunk, 0)
            return 0

        lax.fori_loop(0, a, winner_sweep, 0)

        nrmax, nridx = lax.fori_loop(0, nchunks, argmax_chunk,
                                     (neg16, big16))
        return t_next, nrmax, nridx

    lax.while_loop(lambda c: c[0] < MAX_DET, round_body,
                   (jnp.int32(0),) + carry0)

    @pl.when(s == 0)
    def _():
        pltpu.sync_copy(out_v, out_h)


@jax.jit
def kernel(boxes, scores):
    pad = NPAD - N
    x1 = jnp.pad(boxes[:, 0], (0, pad))
    y1 = jnp.pad(boxes[:, 1], (0, pad))
    x2 = jnp.pad(boxes[:, 2], (0, pad))
    y2 = jnp.pad(boxes[:, 3], (0, pad))
    sc = jnp.pad(scores, (0, pad), constant_values=-1.0)

    nms = functools.partial(
        pl.kernel,
        out_type=jax.ShapeDtypeStruct((MAX_DET * LANES,), jnp.float32),
        mesh=plsc.VectorSubcoreMesh(
            core_axis_name="c", subcore_axis_name="s", num_cores=1),
        compiler_params=pltpu.CompilerParams(needs_layout_passes=False),
        scratch_types=[
            pltpu.VMEM((BUF,), jnp.float32),
            pltpu.VMEM((BUF,), jnp.float32),
            pltpu.VMEM((BUF,), jnp.float32),
            pltpu.VMEM((BUF,), jnp.float32),
            pltpu.VMEM((BUF,), jnp.float32),
            pltpu.VMEM((LANES,), jnp.float32),
            pltpu.VMEM((NSUB * LANES,), jnp.float32),
            pltpu.VMEM((MAX_DET * LANES,), jnp.float32),
            pltpu.VMEM_SHARED((NSUB * LANES,), jnp.float32),
        ],
    )(_nms_kernel)
    out = nms(x1, y1, x2, y2, sc)
    return out.reshape(MAX_DET, LANES)[:, :5]


# speedup vs baseline: 1.5864x; 1.5864x over previous
import functools

import jax
import jax.numpy as jnp
from jax import lax
from jax.experimental import pallas as pl
from jax.experimental.pallas import tpu as pltpu
from jax.experimental.pallas import tpu_sc as plsc

N = 20000
SCORE_THRESH = 0.5
IOU_THRESH = 0.8
MAX_DET = 100

NSUB = 16
LANES = 16
PER_SUB = 1280
CHUNKS = PER_SUB // LANES
NPAD = NSUB * PER_SUB

NEG = -1e30
BIG = 1e30


def _nms_kernel(x1_h, y1_h, x2_h, y2_h, sc_h, out_h,
                x1_v, y1_v, x2_v, y2_v, sv, cand_v, allc_v, out_v, shared):
    s = lax.axis_index("s")
    base = s * PER_SUB
    base_f = base.astype(jnp.float32)
    io = lax.iota(jnp.int32, LANES)
    iof = io.astype(jnp.float32)
    zeros_i = jnp.zeros((LANES,), jnp.int32)
    neg16 = jnp.full((LANES,), NEG, jnp.float32)
    big16 = jnp.full((LANES,), BIG, jnp.float32)

    pltpu.sync_copy(x1_h.at[pl.ds(base, PER_SUB)], x1_v)
    pltpu.sync_copy(y1_h.at[pl.ds(base, PER_SUB)], y1_v)
    pltpu.sync_copy(x2_h.at[pl.ds(base, PER_SUB)], x2_v)
    pltpu.sync_copy(y2_h.at[pl.ds(base, PER_SUB)], y2_v)
    pltpu.sync_copy(sc_h.at[pl.ds(base, PER_SUB)], sv)

    def init_chunk(k, carry):
        rmax, ridx = carry
        sl = pl.ds(k * LANES, LANES)
        v = sv[sl]
        v = jnp.where(v > SCORE_THRESH, v, NEG)
        sv[sl] = v
        gidx = base_f + (k * LANES).astype(jnp.float32) + iof
        ridx = jnp.where(v > rmax, gidx, ridx)
        rmax = jnp.maximum(rmax, v)
        return rmax, ridx

    carry0 = plsc.parallel_loop(
        0, CHUNKS, unroll=8, carry=(neg16, big16))(init_chunk)

    def pick(t, carry):
        rmax, ridx = carry
        smax = jnp.max(rmax)
        lidx = jnp.min(jnp.where(rmax == smax, ridx, big16))
        li = jnp.clip(lidx - base_f, 0.0, float(PER_SUB - 1)).astype(jnp.int32)
        liv = zeros_i + li
        gx1 = plsc.load_gather(x1_v, [liv])
        gy1 = plsc.load_gather(y1_v, [liv])
        gx2 = plsc.load_gather(x2_v, [liv])
        gy2 = plsc.load_gather(y2_v, [liv])
        row = (jnp.where(io == 0, smax, 0.0)
               + jnp.where(io == 1, lidx, 0.0)
               + jnp.where(io == 2, gx1, 0.0)
               + jnp.where(io == 3, gy1, 0.0)
               + jnp.where(io == 4, gx2, 0.0)
               + jnp.where(io == 5, gy2, 0.0))
        cand_v[...] = row
        pltpu.sync_copy(cand_v, shared.at[pl.ds(s * LANES, LANES)])
        plsc.subcore_barrier()
        pltpu.sync_copy(shared, allc_v)
        plsc.subcore_barrier()
        stride = io * LANES
        scv = plsc.load_gather(allc_v, [stride])
        idv = plsc.load_gather(allc_v, [stride + 1])
        gmax = jnp.max(scv)
        ok = gmax > NEG
        widx = jnp.min(jnp.where(scv == gmax, idv, big16))
        cw = jnp.min(jnp.where((scv == gmax) & (idv == widx), iof, big16)
                     ).astype(jnp.int32)
        cwv = zeros_i + cw * LANES
        bx1 = plsc.load_gather(allc_v, [cwv + 2])
        by1 = plsc.load_gather(allc_v, [cwv + 3])
        bx2 = plsc.load_gather(allc_v, [cwv + 4])
        by2 = plsc.load_gather(allc_v, [cwv + 5])

        @pl.when(s == 0)
        def _():
            orow = (jnp.where(io == 0, bx1, 0.0)
                    + jnp.where(io == 1, by1, 0.0)
                    + jnp.where(io == 2, bx2, 0.0)
                    + jnp.where(io == 3, by2, 0.0)
                    + jnp.where(io == 4, gmax, 0.0))
            out_v[pl.ds(t * LANES, LANES)] = orow * jnp.where(ok, 1.0, 0.0)

        area_a = (bx2 - bx1) * (by2 - by1)

        def supp_chunk(k, carry):
            nrun, nidx = carry
            sl = pl.ds(k * LANES, LANES)
            sv_k = sv[sl]
            x1k = x1_v[sl]
            y1k = y1_v[sl]
            x2k = x2_v[sl]
            y2k = y2_v[sl]
            xx1 = jnp.maximum(bx1, x1k)
            yy1 = jnp.maximum(by1, y1k)
            xx2 = jnp.minimum(bx2, x2k)
            yy2 = jnp.minimum(by2, y2k)
            inter = jnp.maximum(xx2 - xx1, 0.0) * jnp.maximum(yy2 - yy1, 0.0)
            area_b = (x2k - x1k) * (y2k - y1k)
            union = area_a + area_b - inter
            iou = inter / jnp.maximum(union, 1e-9)
            supp = (iou > IOU_THRESH) & ok
            s2 = jnp.where(supp, NEG, sv_k)
            sv[sl] = s2
            gidx = base_f + (k * LANES).astype(jnp.float32) + iof
            nidx = jnp.where(s2 > nrun, gidx, nidx)
            nrun = jnp.maximum(nrun, s2)
            return nrun, nidx

        return plsc.parallel_loop(
            0, CHUNKS, unroll=8, carry=(neg16, big16))(supp_chunk)

    lax.fori_loop(0, MAX_DET, pick, carry0)

    @pl.when(s == 0)
    def _():
        pltpu.sync_copy(out_v, out_h)


@jax.jit
def kernel(boxes, scores):
    pad = NPAD - N
    x1 = jnp.pad(boxes[:, 0], (0, pad))
    y1 = jnp.pad(boxes[:, 1], (0, pad))
    x2 = jnp.pad(boxes[:, 2], (0, pad))
    y2 = jnp.pad(boxes[:, 3], (0, pad))
    sc = jnp.pad(scores, (0, pad), constant_values=-1.0)

    nms = functools.partial(
        pl.kernel,
        out_type=jax.ShapeDtypeStruct((MAX_DET * LANES,), jnp.float32),
        mesh=plsc.VectorSubcoreMesh(
            core_axis_name="c", subcore_axis_name="s", num_cores=1),
        compiler_params=pltpu.CompilerParams(needs_layout_passes=False),
        scratch_types=[
            pltpu.VMEM((PER_SUB,), jnp.float32),
            pltpu.VMEM((PER_SUB,), jnp.float32),
            pltpu.VMEM((PER_SUB,), jnp.float32),
            pltpu.VMEM((PER_SUB,), jnp.float32),
            pltpu.VMEM((PER_SUB,), jnp.float32),
            pltpu.VMEM((LANES,), jnp.float32),
            pltpu.VMEM((NSUB * LANES,), jnp.float32),
            pltpu.VMEM((MAX_DET * LANES,), jnp.float32),
            pltpu.VMEM_SHARED((NSUB * LANES,), jnp.float32),
        ],
    )(_nms_kernel)
    out = nms(x1, y1, x2, y2, sc)
    return out.reshape(MAX_DET, LANES)[:, :5]


# speedup vs baseline: 1.6576x; 1.0449x over previous
import functools

import jax
import jax.numpy as jnp
from jax import lax
from jax.experimental import pallas as pl
from jax.experimental.pallas import tpu as pltpu
from jax.experimental.pallas import tpu_sc as plsc

N = 20000
SCORE_THRESH = 0.5
IOU_THRESH = 0.8
MAX_DET = 100

NSUB = 16
LANES = 16
PER_SUB = 1280
CHUNKS = PER_SUB // LANES
NPAD = NSUB * PER_SUB

NEG = -1e30
BIG = 1e30


def _nms_kernel(x1_h, y1_h, x2_h, y2_h, sc_h, out_h,
                x1_v, y1_v, x2_v, y2_v, sv, cand_v, allc_v, out_v, shared):
    s = lax.axis_index("s")
    base = s * PER_SUB
    base_f = base.astype(jnp.float32)
    io = lax.iota(jnp.int32, LANES)
    iof = io.astype(jnp.float32)
    zeros_i = jnp.zeros((LANES,), jnp.int32)
    neg16 = jnp.full((LANES,), NEG, jnp.float32)
    big16 = jnp.full((LANES,), BIG, jnp.float32)

    pltpu.sync_copy(x1_h.at[pl.ds(base, PER_SUB)], x1_v)
    pltpu.sync_copy(y1_h.at[pl.ds(base, PER_SUB)], y1_v)
    pltpu.sync_copy(x2_h.at[pl.ds(base, PER_SUB)], x2_v)
    pltpu.sync_copy(y2_h.at[pl.ds(base, PER_SUB)], y2_v)
    pltpu.sync_copy(sc_h.at[pl.ds(base, PER_SUB)], sv)

    def init_chunk(k, carry):
        rmax, ridx = carry
        sl = pl.ds(k * LANES, LANES)
        v = sv[sl]
        v = jnp.where(v > SCORE_THRESH, v, NEG)
        sv[sl] = v
        gidx = base_f + (k * LANES).astype(jnp.float32) + iof
        ridx = jnp.where(v > rmax, gidx, ridx)
        rmax = jnp.maximum(rmax, v)
        return rmax, ridx

    carry0 = plsc.parallel_loop(
        0, CHUNKS, unroll=8, carry=(neg16, big16))(init_chunk)

    def pick(t, carry):
        rmax, ridx = carry
        smax = jnp.max(rmax)
        lidx = jnp.min(jnp.where(rmax == smax, ridx, big16))
        li = jnp.clip(lidx - base_f, 0.0, float(PER_SUB - 1)).astype(jnp.int32)
        liv = zeros_i + li
        gx1 = plsc.load_gather(x1_v, [liv])
        gy1 = plsc.load_gather(y1_v, [liv])
        gx2 = plsc.load_gather(x2_v, [liv])
        gy2 = plsc.load_gather(y2_v, [liv])
        key = lidx * float(LANES) + s.astype(jnp.float32)
        row = (jnp.where(io == 0, smax, 0.0)
               + jnp.where(io == 1, key, 0.0)
               + jnp.where(io == 2, gx1, 0.0)
               + jnp.where(io == 3, gy1, 0.0)
               + jnp.where(io == 4, gx2, 0.0)
               + jnp.where(io == 5, gy2, 0.0))
        cand_v[...] = row
        off = (t & 1) * (NSUB * LANES)
        pltpu.sync_copy(cand_v, shared.at[pl.ds(off + s * LANES, LANES)])
        plsc.subcore_barrier()
        pltpu.sync_copy(shared.at[pl.ds(off, NSUB * LANES)], allc_v)
        stride = io * LANES
        scv = plsc.load_gather(allc_v, [stride])
        keyv = plsc.load_gather(allc_v, [stride + 1])
        gmax = jnp.max(scv)
        ok = gmax > NEG
        cw = (jnp.min(jnp.where(scv == gmax, keyv, big16)).astype(jnp.int32)
              & (LANES - 1))
        cwv = zeros_i + cw * LANES
        bx1 = plsc.load_gather(allc_v, [cwv + 2])
        by1 = plsc.load_gather(allc_v, [cwv + 3])
        bx2 = plsc.load_gather(allc_v, [cwv + 4])
        by2 = plsc.load_gather(allc_v, [cwv + 5])

        @pl.when(s == 0)
        def _():
            orow = (jnp.where(io == 0, bx1, 0.0)
                    + jnp.where(io == 1, by1, 0.0)
                    + jnp.where(io == 2, bx2, 0.0)
                    + jnp.where(io == 3, by2, 0.0)
                    + jnp.where(io == 4, gmax, 0.0))
            out_v[pl.ds(t * LANES, LANES)] = orow * jnp.where(ok, 1.0, 0.0)

        area_a = (bx2 - bx1) * (by2 - by1)

        def supp_chunk(k, carry):
            nrun, nidx = carry
            sl = pl.ds(k * LANES, LANES)
            sv_k = sv[sl]
            x1k = x1_v[sl]
            y1k = y1_v[sl]
            x2k = x2_v[sl]
            y2k = y2_v[sl]
            xx1 = jnp.maximum(bx1, x1k)
            yy1 = jnp.maximum(by1, y1k)
            xx2 = jnp.minimum(bx2, x2k)
            yy2 = jnp.minimum(by2, y2k)
            inter = jnp.maximum(xx2 - xx1, 0.0) * jnp.maximum(yy2 - yy1, 0.0)
            area_b = (x2k - x1k) * (y2k - y1k)
            union = area_a + area_b - inter
            iou = inter / jnp.maximum(union, 1e-9)
            supp = (iou > IOU_THRESH) & ok
            s2 = jnp.where(supp, NEG, sv_k)
            sv[sl] = s2
            gidx = base_f + (k * LANES).astype(jnp.float32) + iof
            nidx = jnp.where(s2 > nrun, gidx, nidx)
            nrun = jnp.maximum(nrun, s2)
            return nrun, nidx

        return plsc.parallel_loop(
            0, CHUNKS, unroll=8, carry=(neg16, big16))(supp_chunk)

    lax.fori_loop(0, MAX_DET, pick, carry0)

    @pl.when(s == 0)
    def _():
        pltpu.sync_copy(out_v, out_h)


@jax.jit
def kernel(boxes, scores):
    pad = NPAD - N
    x1 = jnp.pad(boxes[:, 0], (0, pad))
    y1 = jnp.pad(boxes[:, 1], (0, pad))
    x2 = jnp.pad(boxes[:, 2], (0, pad))
    y2 = jnp.pad(boxes[:, 3], (0, pad))
    sc = jnp.pad(scores, (0, pad), constant_values=-1.0)

    nms = functools.partial(
        pl.kernel,
        out_type=jax.ShapeDtypeStruct((MAX_DET * LANES,), jnp.float32),
        mesh=plsc.VectorSubcoreMesh(
            core_axis_name="c", subcore_axis_name="s", num_cores=1),
        compiler_params=pltpu.CompilerParams(needs_layout_passes=False),
        scratch_types=[
            pltpu.VMEM((PER_SUB,), jnp.float32),
            pltpu.VMEM((PER_SUB,), jnp.float32),
            pltpu.VMEM((PER_SUB,), jnp.float32),
            pltpu.VMEM((PER_SUB,), jnp.float32),
            pltpu.VMEM((PER_SUB,), jnp.float32),
            pltpu.VMEM((LANES,), jnp.float32),
            pltpu.VMEM((NSUB * LANES,), jnp.float32),
            pltpu.VMEM((MAX_DET * LANES,), jnp.float32),
            pltpu.VMEM_SHARED((2 * NSUB * LANES,), jnp.float32),
        ],
    )(_nms_kernel)
    out = nms(x1, y1, x2, y2, sc)
    return out.reshape(MAX_DET, LANES)[:, :5]


# speedup vs baseline: 3.6570x; 2.2062x over previous
import functools

import jax
import jax.numpy as jnp
from jax import lax
from jax.experimental import pallas as pl
from jax.experimental.pallas import tpu as pltpu
from jax.experimental.pallas import tpu_sc as plsc

N = 20000
SCORE_THRESH = 0.5
IOU_THRESH = 0.8
MAX_DET = 100

NSUB = 16
LANES = 16
PER_SUB = 1280
CHUNKS = PER_SUB // LANES
NPAD = NSUB * PER_SUB
D = 16
FB = NSUB * D
WCAP = 128

NEG = -1e30
BIG = 1e30


def _nms_kernel(x1_h, y1_h, x2_h, y2_h, sc_h, out_h,
                x1_v, y1_v, x2_v, y2_v, sv, stage_v, cblk_v,
                wx1_a, wy1_a, wx2_a, wy2_a, out_v, shared):
    s = lax.axis_index("s")
    base = s * PER_SUB
    base_f = base.astype(jnp.float32)
    sf = s.astype(jnp.float32)
    io = lax.iota(jnp.int32, LANES)
    iof = io.astype(jnp.float32)
    zeros_i = jnp.zeros((LANES,), jnp.int32)
    neg16 = jnp.full((LANES,), NEG, jnp.float32)
    big16 = jnp.full((LANES,), BIG, jnp.float32)
    z16 = jnp.zeros((LANES,), jnp.float32)

    pltpu.sync_copy(x1_h.at[pl.ds(base, PER_SUB)], x1_v)
    pltpu.sync_copy(y1_h.at[pl.ds(base, PER_SUB)], y1_v)
    pltpu.sync_copy(x2_h.at[pl.ds(base, PER_SUB)], x2_v)
    pltpu.sync_copy(y2_h.at[pl.ds(base, PER_SUB)], y2_v)
    pltpu.sync_copy(sc_h.at[pl.ds(base, PER_SUB)], sv)

    def mask_chunk(k):
        sl = pl.ds(k * LANES, LANES)
        v = sv[sl]
        sv[sl] = jnp.where(v > SCORE_THRESH, v, NEG)

    plsc.parallel_loop(0, CHUNKS, unroll=8)(mask_chunk)

    def iou_of(ax1, ay1, ax2, ay2, area_a, bx1, by1, bx2, by2):
        xx1 = jnp.maximum(ax1, bx1)
        yy1 = jnp.maximum(ay1, by1)
        xx2 = jnp.minimum(ax2, bx2)
        yy2 = jnp.minimum(ay2, by2)
        inter = jnp.maximum(xx2 - xx1, 0.0) * jnp.maximum(yy2 - yy1, 0.0)
        area_b = (bx2 - bx1) * (by2 - by1)
        union = area_a + area_b - inter
        return inter / jnp.maximum(union, 1e-9)

    def round_body(carry):
        (t0,) = carry

        def epass(e, c):
            ext_s, ext_p, ls, lp = c

            def am(k, cc):
                rmax, rpos = cc
                v = sv[pl.ds(k * LANES, LANES)]
                pos = (k * LANES).astype(jnp.float32) + iof
                elig = (v < ls) | ((v == ls) & (pos > lp))
                veff = jnp.where(elig, v, NEG)
                rpos = jnp.where(veff > rmax, pos, rpos)
                rmax = jnp.maximum(rmax, veff)
                return rmax, rpos

            rmax, rpos = plsc.parallel_loop(
                0, CHUNKS, unroll=8, carry=(neg16, big16))(am)
            smax = jnp.max(rmax)
            lpos = jnp.min(jnp.where(rmax == smax, rpos, big16))
            ext_s = jnp.where(io == e, smax, ext_s)
            ext_p = jnp.where(io == e, lpos, ext_p)
            return ext_s, ext_p, smax, lpos

        ext_s, ext_p, _, _ = lax.fori_loop(
            0, D, epass, (neg16, big16, jnp.float32(BIG), jnp.float32(-1.0)))

        ext_pi = jnp.clip(ext_p, 0.0, float(PER_SUB - 1)).astype(jnp.int32)
        ex1 = plsc.load_gather(x1_v, [ext_pi])
        ey1 = plsc.load_gather(y1_v, [ext_pi])
        ex2 = plsc.load_gather(x2_v, [ext_pi])
        ey2 = plsc.load_gather(y2_v, [ext_pi])
        keyv = (base_f + ext_p) * float(LANES) + sf
        stage_v[pl.ds(0, LANES)] = ext_s
        stage_v[pl.ds(LANES, LANES)] = keyv
        stage_v[pl.ds(2 * LANES, LANES)] = ex1
        stage_v[pl.ds(3 * LANES, LANES)] = ey1
        stage_v[pl.ds(4 * LANES, LANES)] = ex2
        stage_v[pl.ds(5 * LANES, LANES)] = ey2
        for f in range(6):
            pltpu.sync_copy(stage_v.at[pl.ds(f * LANES, LANES)],
                            shared.at[pl.ds(f * FB + s * D, LANES)])
        plsc.subcore_barrier()
        pltpu.sync_copy(shared, cblk_v)
        plsc.subcore_barrier()

        b_s = plsc.load_gather(cblk_v, [io * D + (D - 1)])

        cvi = io * D
        cur_s = plsc.load_gather(cblk_v, [cvi])
        cur_k = plsc.load_gather(cblk_v, [cvi + FB])
        cx1 = plsc.load_gather(cblk_v, [cvi + 2 * FB])
        cy1 = plsc.load_gather(cblk_v, [cvi + 3 * FB])
        cx2 = plsc.load_gather(cblk_v, [cvi + 4 * FB])
        cy2 = plsc.load_gather(cblk_v, [cvi + 5 * FB])
        cmax0 = jnp.max(cur_s)
        go0 = cmax0 > NEG
        cont0 = go0 & (t0 < MAX_DET)

        def acc(c):
            (cur_s, cur_k, cx1, cy1, cx2, cy2, cvi, exb, a, t, cmax,
             cont) = c
            ck = jnp.min(jnp.where(cur_s == cmax, cur_k, big16))
            cw = ck.astype(jnp.int32) & (LANES - 1)
            wvi = jnp.max(jnp.where(io == cw, cvi, zeros_i))
            wviv = zeros_i + wvi
            wx1 = plsc.load_gather(cblk_v, [wviv + 2 * FB])
            wy1 = plsc.load_gather(cblk_v, [wviv + 3 * FB])
            wx2 = plsc.load_gather(cblk_v, [wviv + 4 * FB])
            wy2 = plsc.load_gather(cblk_v, [wviv + 5 * FB])
            av = zeros_i + a
            plsc.store_scatter(wx1_a, [av], wx1)
            plsc.store_scatter(wy1_a, [av], wy1)
            plsc.store_scatter(wx2_a, [av], wx2)
            plsc.store_scatter(wy2_a, [av], wy2)

            @pl.when(s == 0)
            def _():
                orow = (jnp.where(io == 0, wx1, 0.0)
                        + jnp.where(io == 1, wy1, 0.0)
                        + jnp.where(io == 2, wx2, 0.0)
                        + jnp.where(io == 3, wy2, 0.0)
                        + jnp.where(io == 4, cmax, 0.0))
                out_v[pl.ds(t * LANES, LANES)] = orow

            area_w = (wx2 - wx1) * (wy2 - wy1)
            iou_c = iou_of(wx1, wy1, wx2, wy2, area_w, cx1, cy1, cx2, cy2)
            deadc = (iou_c > IOU_THRESH) & (cur_s > NEG)
            otherdead = jnp.max(jnp.where(deadc & (io != cw), 1.0, 0.0))
            nwvi = wvi + 1
            exh = (nwvi & (D - 1)) == 0
            gv = zeros_i + jnp.where(exh, wvi, nwvi)
            ns = jnp.where(exh, NEG, plsc.load_gather(cblk_v, [gv]))
            nk = plsc.load_gather(cblk_v, [gv + FB])
            nx1 = plsc.load_gather(cblk_v, [gv + 2 * FB])
            ny1 = plsc.load_gather(cblk_v, [gv + 3 * FB])
            nx2 = plsc.load_gather(cblk_v, [gv + 4 * FB])
            ny2 = plsc.load_gather(cblk_v, [gv + 5 * FB])
            area_n = (nx2 - nx1) * (ny2 - ny1)
            iou_nw = iou_of(nx1, ny1, nx2, ny2, area_n, wx1, wy1, wx2, wy2)
            dinit = jnp.where(iou_nw > IOU_THRESH, 1.0, 0.0)

            def wchunk(ci, dacc):
                wl = pl.ds(ci * LANES, LANES)
                qx1 = wx1_a[wl]
                qy1 = wy1_a[wl]
                qx2 = wx2_a[wl]
                qy2 = wy2_a[wl]
                iou_q = iou_of(nx1, ny1, nx2, ny2, area_n,
                               qx1, qy1, qx2, qy2)
                valid = (ci * LANES + io) < a
                return jnp.maximum(
                    dacc, jnp.where(valid & (iou_q > IOU_THRESH), 1.0, 0.0))

            dvec = lax.fori_loop(0, (a + LANES - 1) // LANES, wchunk, dinit)
            ndead = jnp.max(jnp.where(ns > NEG, dvec, 0.0)) > 0.5
            sel = io == cw
            cur_s = jnp.where(sel, ns, cur_s)
            cur_k = jnp.where(sel, nk, cur_k)
            cx1 = jnp.where(sel, nx1, cx1)
            cy1 = jnp.where(sel, ny1, cy1)
            cx2 = jnp.where(sel, nx2, cx2)
            cy2 = jnp.where(sel, ny2, cy2)
            cvi = jnp.where(sel, zeros_i + nwvi, cvi)
            exb = jnp.where(sel & exh, b_s, exb)
            a = a + 1
            t = t + 1
            cmax2 = jnp.max(cur_s)
            maxb2 = jnp.max(exb)
            cont2 = ((cmax2 > NEG) & (cmax2 > maxb2) & (otherdead < 0.5)
                     & jnp.logical_not(ndead) & (t < MAX_DET))
            return (cur_s, cur_k, cx1, cy1, cx2, cy2, cvi, exb, a, t,
                    cmax2, cont2)

        fin = lax.while_loop(
            lambda c: c[-1], acc,
            (cur_s, cur_k, cx1, cy1, cx2, cy2, cvi, neg16, jnp.int32(0),
             t0, cmax0, cont0))
        a_f = fin[8]
        t_f = fin[9]

        @pl.when(jnp.logical_not(go0) & (s == 0))
        def _():
            def zfill(tt, _):
                out_v[pl.ds(tt * LANES, LANES)] = z16
                return 0

            lax.fori_loop(t0, MAX_DET, zfill, 0)

        t_next = jnp.where(go0, t_f, jnp.int32(MAX_DET))

        @pl.when(t_next < MAX_DET)
        def _():
            pl.delay(500)

            def wsweep(ai, _):
                aiv = zeros_i + ai
                qx1 = plsc.load_gather(wx1_a, [aiv])
                qy1 = plsc.load_gather(wy1_a, [aiv])
                qx2 = plsc.load_gather(wx2_a, [aiv])
                qy2 = plsc.load_gather(wy2_a, [aiv])
                area_q = (qx2 - qx1) * (qy2 - qy1)

                def sch(k):
                    sl = pl.ds(k * LANES, LANES)
                    v = sv[sl]
                    x1k = x1_v[sl]
                    y1k = y1_v[sl]
                    x2k = x2_v[sl]
                    y2k = y2_v[sl]
                    iou = iou_of(qx1, qy1, qx2, qy2, area_q,
                                 x1k, y1k, x2k, y2k)
                    sv[sl] = jnp.where(iou > IOU_THRESH, NEG, v)

                plsc.parallel_loop(0, CHUNKS, unroll=8)(sch)
                return 0

            lax.fori_loop(0, a_f, wsweep, 0)

        return (t_next,)

    lax.while_loop(lambda c: c[0] < MAX_DET, round_body, (jnp.int32(0),))

    @pl.when(s == 0)
    def _():
        pltpu.sync_copy(out_v, out_h)


@jax.jit
def kernel(boxes, scores):
    pad = NPAD - N
    x1 = jnp.pad(boxes[:, 0], (0, pad))
    y1 = jnp.pad(boxes[:, 1], (0, pad))
    x2 = jnp.pad(boxes[:, 2], (0, pad))
    y2 = jnp.pad(boxes[:, 3], (0, pad))
    sc = jnp.pad(scores, (0, pad), constant_values=-1.0)

    nms = functools.partial(
        pl.kernel,
        out_type=jax.ShapeDtypeStruct((MAX_DET * LANES,), jnp.float32),
        mesh=plsc.VectorSubcoreMesh(
            core_axis_name="c", subcore_axis_name="s", num_cores=1),
        compiler_params=pltpu.CompilerParams(needs_layout_passes=False),
        scratch_types=[
            pltpu.VMEM((PER_SUB,), jnp.float32),
            pltpu.VMEM((PER_SUB,), jnp.float32),
            pltpu.VMEM((PER_SUB,), jnp.float32),
            pltpu.VMEM((PER_SUB,), jnp.float32),
            pltpu.VMEM((PER_SUB,), jnp.float32),
            pltpu.VMEM((6 * LANES,), jnp.float32),
            pltpu.VMEM((6 * FB,), jnp.float32),
            pltpu.VMEM((WCAP,), jnp.float32),
            pltpu.VMEM((WCAP,), jnp.float32),
            pltpu.VMEM((WCAP,), jnp.float32),
            pltpu.VMEM((WCAP,), jnp.float32),
            pltpu.VMEM((MAX_DET * LANES,), jnp.float32),
            pltpu.VMEM_SHARED((6 * FB,), jnp.float32),
        ],
    )(_nms_kernel)
    out = nms(x1, y1, x2, y2, sc)
    return out.reshape(MAX_DET, LANES)[:, :5]


# speedup vs baseline: 3.8017x; 1.0396x over previous
import functools

import jax
import jax.numpy as jnp
from jax import lax
from jax.experimental import pallas as pl
from jax.experimental.pallas import tpu as pltpu
from jax.experimental.pallas import tpu_sc as plsc

N = 20000
SCORE_THRESH = 0.5
IOU_THRESH = 0.8
MAX_DET = 100

NSUB = 16
LANES = 16
PER_SUB = 1280
CHUNKS = PER_SUB // LANES
NPAD = NSUB * PER_SUB
D = 16
FB = NSUB * D
WCAP = 128

NEG = -1e30
BIG = 1e30


def _nms_kernel(x1_h, y1_h, x2_h, y2_h, sc_h, out_h,
                x1_v, y1_v, x2_v, y2_v, sv, stage_v, cblk_v,
                wx1_a, wy1_a, wx2_a, wy2_a, out_v, shared):
    s = lax.axis_index("s")
    base = s * PER_SUB
    base_f = base.astype(jnp.float32)
    sf = s.astype(jnp.float32)
    io = lax.iota(jnp.int32, LANES)
    iof = io.astype(jnp.float32)
    zeros_i = jnp.zeros((LANES,), jnp.int32)
    neg16 = jnp.full((LANES,), NEG, jnp.float32)
    big16 = jnp.full((LANES,), BIG, jnp.float32)
    z16 = jnp.zeros((LANES,), jnp.float32)

    pltpu.sync_copy(x1_h.at[pl.ds(base, PER_SUB)], x1_v)
    pltpu.sync_copy(y1_h.at[pl.ds(base, PER_SUB)], y1_v)
    pltpu.sync_copy(x2_h.at[pl.ds(base, PER_SUB)], x2_v)
    pltpu.sync_copy(y2_h.at[pl.ds(base, PER_SUB)], y2_v)
    pltpu.sync_copy(sc_h.at[pl.ds(base, PER_SUB)], sv)

    def mask_chunk(k):
        sl = pl.ds(k * LANES, LANES)
        v = sv[sl]
        sv[sl] = jnp.where(v > SCORE_THRESH, v, NEG)

    plsc.parallel_loop(0, CHUNKS, unroll=8)(mask_chunk)

    def iou_of(ax1, ay1, ax2, ay2, area_a, bx1, by1, bx2, by2):
        xx1 = jnp.maximum(ax1, bx1)
        yy1 = jnp.maximum(ay1, by1)
        xx2 = jnp.minimum(ax2, bx2)
        yy2 = jnp.minimum(ay2, by2)
        inter = jnp.maximum(xx2 - xx1, 0.0) * jnp.maximum(yy2 - yy1, 0.0)
        area_b = (bx2 - bx1) * (by2 - by1)
        union = area_a + area_b - inter
        return inter / jnp.maximum(union, 1e-9)

    def round_body(carry):
        (t0,) = carry

        def epass(e, c):
            ext_s, ext_p, ls, lp = c

            def am(k, cc):
                rmax, rpos = cc
                v = sv[pl.ds(k * LANES, LANES)]
                pos = (k * LANES).astype(jnp.float32) + iof
                elig = (v < ls) | ((v == ls) & (pos > lp))
                veff = jnp.where(elig, v, NEG)
                rpos = jnp.where(veff > rmax, pos, rpos)
                rmax = jnp.maximum(rmax, veff)
                return rmax, rpos

            rmax, rpos = plsc.parallel_loop(
                0, CHUNKS, unroll=8, carry=(neg16, big16))(am)
            smax = jnp.max(rmax)
            lpos = jnp.min(jnp.where(rmax == smax, rpos, big16))
            ext_s = jnp.where(io == e, smax, ext_s)
            ext_p = jnp.where(io == e, lpos, ext_p)
            return ext_s, ext_p, smax, lpos

        ext_s, ext_p, _, _ = lax.fori_loop(
            0, D, epass, (neg16, big16, jnp.float32(BIG), jnp.float32(-1.0)))

        ext_pi = jnp.clip(ext_p, 0.0, float(PER_SUB - 1)).astype(jnp.int32)
        ex1 = plsc.load_gather(x1_v, [ext_pi])
        ey1 = plsc.load_gather(y1_v, [ext_pi])
        ex2 = plsc.load_gather(x2_v, [ext_pi])
        ey2 = plsc.load_gather(y2_v, [ext_pi])
        keyv = ((base_f + ext_p) * float(LANES) + sf) * float(D) + iof
        stage_v[pl.ds(0, LANES)] = ext_s
        stage_v[pl.ds(LANES, LANES)] = keyv
        stage_v[pl.ds(2 * LANES, LANES)] = ex1
        stage_v[pl.ds(3 * LANES, LANES)] = ey1
        stage_v[pl.ds(4 * LANES, LANES)] = ex2
        stage_v[pl.ds(5 * LANES, LANES)] = ey2
        for f in range(6):
            pltpu.sync_copy(stage_v.at[pl.ds(f * LANES, LANES)],
                            shared.at[pl.ds(f * FB + s * D, LANES)])
        plsc.subcore_barrier()
        pltpu.sync_copy(shared, cblk_v)
        plsc.subcore_barrier()

        b_s = plsc.load_gather(cblk_v, [io * D + (D - 1)])

        cvi = io * D
        cur_s = plsc.load_gather(cblk_v, [cvi])
        cur_k = plsc.load_gather(cblk_v, [cvi + FB])
        cx1 = plsc.load_gather(cblk_v, [cvi + 2 * FB])
        cy1 = plsc.load_gather(cblk_v, [cvi + 3 * FB])
        cx2 = plsc.load_gather(cblk_v, [cvi + 4 * FB])
        cy2 = plsc.load_gather(cblk_v, [cvi + 5 * FB])
        cmax0 = jnp.max(cur_s)
        go0 = cmax0 > NEG
        cont0 = go0 & (t0 < MAX_DET)

        def acc(c):
            (cur_s, cur_k, cx1, cy1, cx2, cy2, exb, a, t, cmax,
             cont) = c
            ck = jnp.min(jnp.where(cur_s == cmax, cur_k, big16))
            cki = ck.astype(jnp.int32)
            cw = lax.shift_right_logical(cki, 4) & (LANES - 1)
            wvi = cw * D + (cki & (D - 1))
            wviv = zeros_i + wvi
            wx1 = plsc.load_gather(cblk_v, [wviv + 2 * FB])
            wy1 = plsc.load_gather(cblk_v, [wviv + 3 * FB])
            wx2 = plsc.load_gather(cblk_v, [wviv + 4 * FB])
            wy2 = plsc.load_gather(cblk_v, [wviv + 5 * FB])
            av = zeros_i + a
            plsc.store_scatter(wx1_a, [av], wx1)
            plsc.store_scatter(wy1_a, [av], wy1)
            plsc.store_scatter(wx2_a, [av], wx2)
            plsc.store_scatter(wy2_a, [av], wy2)

            @pl.when(s == 0)
            def _():
                orow = (jnp.where(io == 0, wx1, 0.0)
                        + jnp.where(io == 1, wy1, 0.0)
                        + jnp.where(io == 2, wx2, 0.0)
                        + jnp.where(io == 3, wy2, 0.0)
                        + jnp.where(io == 4, cmax, 0.0))
                out_v[pl.ds(t * LANES, LANES)] = orow

            area_w = (wx2 - wx1) * (wy2 - wy1)
            iou_c = iou_of(wx1, wy1, wx2, wy2, area_w, cx1, cy1, cx2, cy2)
            deadc = (iou_c > IOU_THRESH) & (cur_s > NEG)
            otherdead = jnp.max(jnp.where(deadc & (io != cw), 1.0, 0.0))
            nwvi = wvi + 1
            exh = (nwvi & (D - 1)) == 0
            gv = zeros_i + jnp.where(exh, wvi, nwvi)
            ns = jnp.where(exh, NEG, plsc.load_gather(cblk_v, [gv]))
            nk = plsc.load_gather(cblk_v, [gv + FB])
            nx1 = plsc.load_gather(cblk_v, [gv + 2 * FB])
            ny1 = plsc.load_gather(cblk_v, [gv + 3 * FB])
            nx2 = plsc.load_gather(cblk_v, [gv + 4 * FB])
            ny2 = plsc.load_gather(cblk_v, [gv + 5 * FB])
            area_n = (nx2 - nx1) * (ny2 - ny1)
            iou_nw = iou_of(nx1, ny1, nx2, ny2, area_n, wx1, wy1, wx2, wy2)
            dinit = jnp.where(iou_nw > IOU_THRESH, 1.0, 0.0)

            def wchunk(ci, dacc):
                wl = pl.ds(ci * LANES, LANES)
                qx1 = wx1_a[wl]
                qy1 = wy1_a[wl]
                qx2 = wx2_a[wl]
                qy2 = wy2_a[wl]
                iou_q = iou_of(nx1, ny1, nx2, ny2, area_n,
                               qx1, qy1, qx2, qy2)
                valid = (ci * LANES + io) < a
                return jnp.maximum(
                    dacc, jnp.where(valid & (iou_q > IOU_THRESH), 1.0, 0.0))

            dvec = lax.fori_loop(0, (a + LANES - 1) // LANES, wchunk, dinit)
            ndead = jnp.max(jnp.where(ns > NEG, dvec, 0.0)) > 0.5
            sel = io == cw
            cur_s = jnp.where(sel, ns, cur_s)
            cur_k = jnp.where(sel, nk, cur_k)
            cx1 = jnp.where(sel, nx1, cx1)
            cy1 = jnp.where(sel, ny1, cy1)
            cx2 = jnp.where(sel, nx2, cx2)
            cy2 = jnp.where(sel, ny2, cy2)
            exb = jnp.where(sel & exh, b_s, exb)
            a = a + 1
            t = t + 1
            cmax2 = jnp.max(cur_s)
            maxb2 = jnp.max(exb)
            cont2 = ((cmax2 > NEG) & (cmax2 > maxb2) & (otherdead < 0.5)
                     & jnp.logical_not(ndead) & (t < MAX_DET))
            return (cur_s, cur_k, cx1, cy1, cx2, cy2, exb, a, t,
                    cmax2, cont2)

        fin = lax.while_loop(
            lambda c: c[-1], acc,
            (cur_s, cur_k, cx1, cy1, cx2, cy2, neg16, jnp.int32(0),
             t0, cmax0, cont0))
        a_f = fin[7]
        t_f = fin[8]

        @pl.when(jnp.logical_not(go0) & (s == 0))
        def _():
            def zfill(tt, _):
                out_v[pl.ds(tt * LANES, LANES)] = z16
                return 0

            lax.fori_loop(t0, MAX_DET, zfill, 0)

        t_next = jnp.where(go0, t_f, jnp.int32(MAX_DET))

        @pl.when(t_next < MAX_DET)
        def _():
            pl.delay(500)

            def wsweep(ai, _):
                aiv = zeros_i + ai
                qx1 = plsc.load_gather(wx1_a, [aiv])
                qy1 = plsc.load_gather(wy1_a, [aiv])
                qx2 = plsc.load_gather(wx2_a, [aiv])
                qy2 = plsc.load_gather(wy2_a, [aiv])
                area_q = (qx2 - qx1) * (qy2 - qy1)

                def sch(k):
                    sl = pl.ds(k * LANES, LANES)
                    v = sv[sl]
                    x1k = x1_v[sl]
                    y1k = y1_v[sl]
                    x2k = x2_v[sl]
                    y2k = y2_v[sl]
                    iou = iou_of(qx1, qy1, qx2, qy2, area_q,
                                 x1k, y1k, x2k, y2k)
                    sv[sl] = jnp.where(iou > IOU_THRESH, NEG, v)

                plsc.parallel_loop(0, CHUNKS, unroll=8)(sch)
                return 0

            lax.fori_loop(0, a_f, wsweep, 0)

        return (t_next,)

    lax.while_loop(lambda c: c[0] < MAX_DET, round_body, (jnp.int32(0),))

    @pl.when(s == 0)
    def _():
        pltpu.sync_copy(out_v, out_h)


@jax.jit
def kernel(boxes, scores):
    pad = NPAD - N
    x1 = jnp.pad(boxes[:, 0], (0, pad))
    y1 = jnp.pad(boxes[:, 1], (0, pad))
    x2 = jnp.pad(boxes[:, 2], (0, pad))
    y2 = jnp.pad(boxes[:, 3], (0, pad))
    sc = jnp.pad(scores, (0, pad), constant_values=-1.0)

    nms = functools.partial(
        pl.kernel,
        out_type=jax.ShapeDtypeStruct((MAX_DET * LANES,), jnp.float32),
        mesh=plsc.VectorSubcoreMesh(
            core_axis_name="c", subcore_axis_name="s", num_cores=1),
        compiler_params=pltpu.CompilerParams(needs_layout_passes=False),
        scratch_types=[
            pltpu.VMEM((PER_SUB,), jnp.float32),
            pltpu.VMEM((PER_SUB,), jnp.float32),
            pltpu.VMEM((PER_SUB,), jnp.float32),
            pltpu.VMEM((PER_SUB,), jnp.float32),
            pltpu.VMEM((PER_SUB,), jnp.float32),
            pltpu.VMEM((6 * LANES,), jnp.float32),
            pltpu.VMEM((6 * FB,), jnp.float32),
            pltpu.VMEM((WCAP,), jnp.float32),
            pltpu.VMEM((WCAP,), jnp.float32),
            pltpu.VMEM((WCAP,), jnp.float32),
            pltpu.VMEM((WCAP,), jnp.float32),
            pltpu.VMEM((MAX_DET * LANES,), jnp.float32),
            pltpu.VMEM_SHARED((6 * FB,), jnp.float32),
        ],
    )(_nms_kernel)
    out = nms(x1, y1, x2, y2, sc)
    return out.reshape(MAX_DET, LANES)[:, :5]
